# Initial kernel scaffold; baseline (speedup 1.0000x reference)
import jax
import jax.numpy as jnp
from jax.experimental import pallas as pl


def kernel(x, evidence, mu, sigma):
    raise NotImplementedError("write your pallas kernel here")


# speedup vs baseline: 6.4492x; 6.4492x over previous
import math

import jax
import jax.numpy as jnp
import numpy as np
from jax.experimental import pallas as pl

_N = 2048
_D = 256
_SIZE = 128
_ROWS = 16

_INV_SQRT_2PI = 1.0 / math.sqrt(2.0 * math.pi)


def _leaf_block(x_ref, sel_ref, mask_ref, evg_ref, mu_ref, sigma_ref,
                out_ref, probs_ref):
    xb = x_ref[...]
    sel = sel_ref[...]
    ev_vals = jnp.dot(xb, sel, preferred_element_type=jnp.float32)

    evg = evg_ref[...] > 0.0
    mu = mu_ref[...]
    sigma = sigma_ref[...]
    inv = _INV_SQRT_2PI / sigma

    z = (ev_vals - mu) / sigma
    pdf = jnp.exp(-0.5 * z * z) * inv

    res = jnp.where(evg, ev_vals, mu)
    probs_ref[...] = jnp.where(evg, pdf, inv)

    out_ref[...] = res[:, :, None] * mask_ref[...][None, :, :]


def kernel(x, evidence, mu, sigma):
    n, d = x.shape
    size = mu.shape[0]

    sel = np.zeros((d, size), dtype=np.float32)
    sel[np.arange(size) * 2, np.arange(size)] = 1.0
    sel = jnp.asarray(sel)
    mask = np.zeros((size, d), dtype=np.float32)
    mask[np.arange(size), np.arange(size) * 2] = 1.0
    mask = jnp.asarray(mask)

    evg = evidence[::2].astype(jnp.float32).reshape(1, size)
    mu2 = mu.reshape(1, size)
    sigma2 = sigma.reshape(1, size)

    grid = (n // _ROWS,)
    out, probs = pl.pallas_call(
        _leaf_block,
        grid=grid,
        in_specs=[
            pl.BlockSpec((_ROWS, d), lambda i: (i, 0)),
            pl.BlockSpec((d, size), lambda i: (0, 0)),
            pl.BlockSpec((size, d), lambda i: (0, 0)),
            pl.BlockSpec((1, size), lambda i: (0, 0)),
            pl.BlockSpec((1, size), lambda i: (0, 0)),
            pl.BlockSpec((1, size), lambda i: (0, 0)),
        ],
        out_specs=[
            pl.BlockSpec((_ROWS, size, d), lambda i: (i, 0, 0)),
            pl.BlockSpec((_ROWS, size), lambda i: (i, 0)),
        ],
        out_shape=[
            jax.ShapeDtypeStruct((n, size, d), x.dtype),
            jax.ShapeDtypeStruct((n, size), x.dtype),
        ],
    )(x, sel, mask, evg, mu2, sigma2)
    return out, probs


# speedup vs baseline: 9.4776x; 1.4696x over previous
import math

import jax
import jax.numpy as jnp
import numpy as np
from jax.experimental import pallas as pl

_N = 2048
_D = 256
_SIZE = 128
_ROWS = 64

_INV_SQRT_2PI = 1.0 / math.sqrt(2.0 * math.pi)


def _leaf_block(x_ref, sel_ref, mask_ref, evg_ref, mu_ref, sigma_ref,
                out_ref, probs_ref):
    xb = x_ref[...]
    sel = sel_ref[...]
    ev_vals = jnp.dot(xb, sel, preferred_element_type=jnp.float32)

    evg = evg_ref[...] > 0.0
    mu = mu_ref[...]
    sigma = sigma_ref[...]
    inv = _INV_SQRT_2PI / sigma

    z = (ev_vals - mu) / sigma
    pdf = jnp.exp(-0.5 * z * z) * inv

    res = jnp.where(evg, ev_vals, mu)
    probs_ref[...] = jnp.where(evg, pdf, inv)

    out_ref[...] = res[:, :, None] * mask_ref[...][None, :, :]


def kernel(x, evidence, mu, sigma):
    n, d = x.shape
    size = mu.shape[0]

    sel = np.zeros((d, size), dtype=np.float32)
    sel[np.arange(size) * 2, np.arange(size)] = 1.0
    sel = jnp.asarray(sel)
    mask = np.zeros((size, d), dtype=np.float32)
    mask[np.arange(size), np.arange(size) * 2] = 1.0
    mask = jnp.asarray(mask)

    evg = evidence[::2].astype(jnp.float32).reshape(1, size)
    mu2 = mu.reshape(1, size)
    sigma2 = sigma.reshape(1, size)

    grid = (n // _ROWS,)
    out, probs = pl.pallas_call(
        _leaf_block,
        grid=grid,
        in_specs=[
            pl.BlockSpec((_ROWS, d), lambda i: (i, 0)),
            pl.BlockSpec((d, size), lambda i: (0, 0)),
            pl.BlockSpec((size, d), lambda i: (0, 0)),
            pl.BlockSpec((1, size), lambda i: (0, 0)),
            pl.BlockSpec((1, size), lambda i: (0, 0)),
            pl.BlockSpec((1, size), lambda i: (0, 0)),
        ],
        out_specs=[
            pl.BlockSpec((_ROWS, size, d), lambda i: (i, 0, 0)),
            pl.BlockSpec((_ROWS, size), lambda i: (i, 0)),
        ],
        out_shape=[
            jax.ShapeDtypeStruct((n, size, d), x.dtype),
            jax.ShapeDtypeStruct((n, size), x.dtype),
        ],
    )(x, sel, mask, evg, mu2, sigma2)
    return out, probs


# speedup vs baseline: 9.5287x; 1.0054x over previous
import math

import jax
import jax.numpy as jnp
import numpy as np
from jax.experimental import pallas as pl

_N = 2048
_D = 256
_SIZE = 128
_ROWS = 128

_INV_SQRT_2PI = 1.0 / math.sqrt(2.0 * math.pi)


def _leaf_block(x_ref, sel_ref, mask_ref, evg_ref, mu_ref, sigma_ref,
                out_ref, probs_ref):
    xb = x_ref[...]
    sel = sel_ref[...]
    ev_vals = jnp.dot(xb, sel, preferred_element_type=jnp.float32)

    evg = evg_ref[...] > 0.0
    mu = mu_ref[...]
    sigma = sigma_ref[...]
    inv = _INV_SQRT_2PI / sigma

    z = (ev_vals - mu) / sigma
    pdf = jnp.exp(-0.5 * z * z) * inv

    res = jnp.where(evg, ev_vals, mu)
    probs_ref[...] = jnp.where(evg, pdf, inv)

    out_ref[...] = res[:, :, None] * mask_ref[...][None, :, :]


def kernel(x, evidence, mu, sigma):
    n, d = x.shape
    size = mu.shape[0]

    sel = np.zeros((d, size), dtype=np.float32)
    sel[np.arange(size) * 2, np.arange(size)] = 1.0
    sel = jnp.asarray(sel)
    mask = np.zeros((size, d), dtype=np.float32)
    mask[np.arange(size), np.arange(size) * 2] = 1.0
    mask = jnp.asarray(mask)

    evg = evidence[::2].astype(jnp.float32).reshape(1, size)
    mu2 = mu.reshape(1, size)
    sigma2 = sigma.reshape(1, size)

    grid = (n // _ROWS,)
    out, probs = pl.pallas_call(
        _leaf_block,
        grid=grid,
        in_specs=[
            pl.BlockSpec((_ROWS, d), lambda i: (i, 0)),
            pl.BlockSpec((d, size), lambda i: (0, 0)),
            pl.BlockSpec((size, d), lambda i: (0, 0)),
            pl.BlockSpec((1, size), lambda i: (0, 0)),
            pl.BlockSpec((1, size), lambda i: (0, 0)),
            pl.BlockSpec((1, size), lambda i: (0, 0)),
        ],
        out_specs=[
            pl.BlockSpec((_ROWS, size, d), lambda i: (i, 0, 0)),
            pl.BlockSpec((_ROWS, size), lambda i: (i, 0)),
        ],
        out_shape=[
            jax.ShapeDtypeStruct((n, size, d), x.dtype),
            jax.ShapeDtypeStruct((n, size), x.dtype),
        ],
    )(x, sel, mask, evg, mu2, sigma2)
    return out, probs
